# baseline (speedup 1.0000x reference)
import jax
import jax.numpy as jnp
from jax import lax
from jax.experimental import pallas as pl
from jax.experimental.pallas import tpu as pltpu
from jax.experimental.pallas import tpu_sc as plsc

SIZE = 64
BATCH = 4
PART = 18
NPAIR = BATCH * PART
GSIZE = 13
PAD = 6
NC, NS = 2, 16
PLANE = SIZE * SIZE
WIN = 20


def _sc_body(coords_hbm, g2_hbm, g3_hbm, zin, o0, o1, o2, o3,
             coordsv, g2v, g3v, zbuf, pbig2, pbig3, zsem, psem):
    wid = lax.axis_index("s") * NC + lax.axis_index("c")
    lane = lax.iota(jnp.int32, 16)
    zero16 = jnp.zeros((16,), jnp.float32)

    sh = [
        pltpu.async_copy(coords_hbm, coordsv, zsem),
        pltpu.async_copy(g2_hbm, g2v, zsem),
        pltpu.async_copy(g3_hbm, g3v, zsem),
        pltpu.async_copy(zin, pbig3, zsem),
        pltpu.async_copy(zin.at[pl.ds(0, 4 * PLANE)], zbuf, zsem),
        pltpu.async_copy(zin.at[pl.ds(0, 3 * PLANE)], pbig2, zsem),
    ]
    for h in sh:
        h.wait()

    def do_pair(pair):
        crow = coordsv[pl.ds(pair * 16, 16)]
        x0 = crow[0]
        y0 = crow[1]
        zz = crow[2]

        sx = 63 - x0
        yc = jnp.clip(y0 - PAD, 0, SIZE - GSIZE)
        syo = yc - y0 + PAD
        zidx3 = zz - 1
        zc = jnp.clip(zidx3 - PAD, 0, SIZE - GSIZE)
        szo = zc - zidx3 + PAD
        bb4 = jnp.minimum(zc // 4, (SIZE - WIN) // 4)
        bb = bb4 * 4
        dz = zc - bb

        zh = []
        for j in range(11):
            blk = j + 5 * (j >= bb4).astype(jnp.int32)
            zh.append(pltpu.async_copy(
                zbuf,
                o3.at[pl.ds((pair * 64 + blk * 4) * PLANE, 4 * PLANE)],
                zsem))
        zidx2 = (zz * 4 + 63) // 64 - 1
        zidx1 = (zz * 2 + 63) // 64 - 1
        zidx0 = (zz + 63) // 64 - 1
        pe2 = jnp.maximum(zidx2, 0)
        pe1 = jnp.maximum(zidx1, 0)
        for k in range(3):
            pk = k + (k >= pe2).astype(jnp.int32)
            zh.append(pltpu.async_copy(
                zbuf.at[pl.ds(0, PLANE)],
                o2.at[pl.ds((pair * 4 + pk) * PLANE, PLANE)], zsem))
        pk1 = (pe1 == 0).astype(jnp.int32)
        zh.append(pltpu.async_copy(
            zbuf.at[pl.ds(0, PLANE)],
            o1.at[pl.ds((pair * 2 + pk1) * PLANE, PLANE)], zsem))

        def body_jz(jz, carry):
            zs = szo + jz
            vz = (zs >= 0) & (zs < GSIZE)
            zcl = jnp.full((16,), jnp.clip(zs, 0, GSIZE - 1), jnp.int32)

            def body_j(j, c2):
                ys = syo + j
                vy = (ys >= 0) & (ys < GSIZE)
                ycl = jnp.full((16,), jnp.clip(ys, 0, GSIZE - 1), jnp.int32)
                scale = jnp.where(vz & vy, 1.0, 0.0).astype(jnp.float32)
                base = ((dz + jz) * SIZE + yc + j) * SIZE
                for i in range(4):
                    cx = sx + i * 16 + lane - 57
                    vxf = jnp.where((cx >= 0) & (cx < GSIZE), scale, 0.0)
                    cxc = jnp.clip(cx, 0, GSIZE - 1)
                    v = plsc.load_gather(
                        g3v, [(zcl * GSIZE + ycl) * 16 + cxc]) * vxf
                    pbig3[pl.ds(base + i * 16, 16)] = v
                return c2

            lax.fori_loop(0, GSIZE, body_j, 0)
            return carry

        lax.fori_loop(0, GSIZE, body_jz, 0)

        v2 = (zidx2 >= 0).astype(jnp.float32)
        v1 = (zidx1 >= 0).astype(jnp.float32)
        v0 = (zidx0 >= 0).astype(jnp.float32)

        def body_j2(j, carry):
            ys = syo + j
            vy = (ys >= 0) & (ys < GSIZE)
            ycl = jnp.full((16,), jnp.clip(ys, 0, GSIZE - 1), jnp.int32)
            yscale = jnp.where(vy, 1.0, 0.0).astype(jnp.float32)
            for r, vr in ((0, v0), (1, v1), (2, v2)):
                rcl = jnp.full((16,), r, jnp.int32)
                scale = yscale * vr
                base = (r * SIZE + yc + j) * SIZE
                for i in range(4):
                    cx = sx + i * 16 + lane - 57
                    vxf = jnp.where((cx >= 0) & (cx < GSIZE), scale, 0.0)
                    cxc = jnp.clip(cx, 0, GSIZE - 1)
                    v = plsc.load_gather(
                        g2v, [(rcl * GSIZE + ycl) * 16 + cxc]) * vxf
                    pbig2[pl.ds(base + i * 16, 16)] = v
            return carry

        lax.fori_loop(0, GSIZE, body_j2, 0)

        ph = [
            pltpu.async_copy(
                pbig3, o3.at[pl.ds((pair * 64 + bb) * PLANE, WIN * PLANE)],
                psem),
            pltpu.async_copy(
                pbig2.at[pl.ds(2 * PLANE, PLANE)],
                o2.at[pl.ds((pair * 4 + pe2) * PLANE, PLANE)], psem),
            pltpu.async_copy(
                pbig2.at[pl.ds(1 * PLANE, PLANE)],
                o1.at[pl.ds((pair * 2 + pe1) * PLANE, PLANE)], psem),
            pltpu.async_copy(
                pbig2.at[pl.ds(0, PLANE)],
                o0.at[pl.ds(pair * PLANE, PLANE)], psem),
        ]
        for h in ph:
            h.wait()

        def rz3(jz, carry):
            def rzj(j, c2):
                base = ((dz + jz) * SIZE + yc + j) * SIZE
                for i in range(4):
                    pbig3[pl.ds(base + i * 16, 16)] = zero16
                return c2
            lax.fori_loop(0, GSIZE, rzj, 0)
            return carry

        lax.fori_loop(0, GSIZE, rz3, 0)

        def rz2(j, carry):
            for r in range(3):
                base = (r * SIZE + yc + j) * SIZE
                for i in range(4):
                    pbig2[pl.ds(base + i * 16, 16)] = zero16
            return carry

        lax.fori_loop(0, GSIZE, rz2, 0)
        return zh

    zh_a = do_pair(wid)
    zh_b = do_pair(wid + 32)

    @pl.when(wid + 64 < NPAIR)
    def _():
        for h in do_pair(wid + 64):
            h.wait()

    for h in zh_a + zh_b:
        h.wait()


@jax.jit
def kernel(coords, g0, g1, g2, g3):
    f32 = jnp.float32
    coords16 = jnp.zeros((NPAIR, 16), jnp.int32)
    coords16 = coords16.at[:, :3].set(coords.reshape(NPAIR, 3))
    g2s = jnp.zeros((3, GSIZE, 16), f32)
    g2s = g2s.at[:, :, :GSIZE].set(jnp.stack([g0[0], g1[0], g2[0]]).astype(f32))
    g3s = jnp.zeros((GSIZE, GSIZE, 16), f32)
    g3s = g3s.at[:, :, :GSIZE].set(g3.astype(f32))

    mesh = plsc.VectorSubcoreMesh(
        core_axis_name="c", subcore_axis_name="s",
        num_cores=NC, num_subcores=NS)
    out_type = [
        jax.ShapeDtypeStruct((NPAIR * 1 * PLANE,), f32),
        jax.ShapeDtypeStruct((NPAIR * 2 * PLANE,), f32),
        jax.ShapeDtypeStruct((NPAIR * 4 * PLANE,), f32),
        jax.ShapeDtypeStruct((NPAIR * 64 * PLANE,), f32),
    ]
    scratch = [
        pltpu.VMEM((NPAIR * 16,), jnp.int32),
        pltpu.VMEM((3 * GSIZE * 16,), f32),
        pltpu.VMEM((GSIZE * GSIZE * 16,), f32),
        pltpu.VMEM((4 * PLANE,), f32),
        pltpu.VMEM((3 * PLANE,), f32),
        pltpu.VMEM((WIN * PLANE,), f32),
        pltpu.SemaphoreType.DMA,
        pltpu.SemaphoreType.DMA,
    ]
    o0, o1, o2, o3 = pl.kernel(
        _sc_body, out_type=out_type, mesh=mesh, scratch_types=scratch,
        compiler_params=pltpu.CompilerParams(needs_layout_passes=False),
    )(coords16.reshape(-1), g2s.reshape(-1), g3s.reshape(-1),
      jnp.zeros((WIN * PLANE,), f32))
    return (
        o0.reshape(BATCH, PART, 1, SIZE, SIZE),
        o1.reshape(BATCH, PART, 2, SIZE, SIZE),
        o2.reshape(BATCH, PART, 4, SIZE, SIZE),
        o3.reshape(BATCH, PART, 64, SIZE, SIZE),
    )


# speedup vs baseline: 1.0140x; 1.0140x over previous
import jax
import jax.numpy as jnp
from jax import lax
from jax.experimental import pallas as pl
from jax.experimental.pallas import tpu as pltpu
from jax.experimental.pallas import tpu_sc as plsc

SIZE = 64
BATCH = 4
PART = 18
NPAIR = BATCH * PART
GSIZE = 13
PAD = 6
NC, NS = 2, 16
PLANE = SIZE * SIZE
WIN = 20


def _sc_body(coords_hbm, g2_hbm, g3_hbm, zin, o0, o1, o2, o3,
             coordsv, g2v, g3v, zbuf, pbig2, pbig3, zsem, psem):
    wid = lax.axis_index("s") * NC + lax.axis_index("c")
    lane = lax.iota(jnp.int32, 16)
    zero16 = jnp.zeros((16,), jnp.float32)

    sh = [
        pltpu.async_copy(coords_hbm, coordsv, zsem),
        pltpu.async_copy(g2_hbm, g2v, zsem),
        pltpu.async_copy(g3_hbm, g3v, zsem),
        pltpu.async_copy(zin, pbig3, zsem),
        pltpu.async_copy(zin.at[pl.ds(0, 4 * PLANE)], zbuf, zsem),
        pltpu.async_copy(zin.at[pl.ds(0, 3 * PLANE)], pbig2, zsem),
    ]
    for h in sh:
        h.wait()

    def do_pair(pair, issue_zeros=True):
        crow = coordsv[pl.ds(pair * 16, 16)]
        x0 = crow[0]
        y0 = crow[1]
        zz = crow[2]

        sx = 63 - x0
        yc = jnp.clip(y0 - PAD, 0, SIZE - GSIZE)
        syo = yc - y0 + PAD
        zidx3 = zz - 1
        zc = jnp.clip(zidx3 - PAD, 0, SIZE - GSIZE)
        szo = zc - zidx3 + PAD
        bb4 = jnp.minimum(zc // 4, (SIZE - WIN) // 4)
        bb = bb4 * 4
        dz = zc - bb

        zidx2 = (zz * 4 + 63) // 64 - 1
        zidx1 = (zz * 2 + 63) // 64 - 1
        zidx0 = (zz + 63) // 64 - 1
        pe2 = jnp.maximum(zidx2, 0)
        pe1 = jnp.maximum(zidx1, 0)
        zh = []
        if issue_zeros:
            for j in range(11):
                blk = j + 5 * (j >= bb4).astype(jnp.int32)
                zh.append(pltpu.async_copy(
                    zbuf,
                    o3.at[pl.ds((pair * 64 + blk * 4) * PLANE, 4 * PLANE)],
                    zsem))
            for k in range(3):
                pk = k + (k >= pe2).astype(jnp.int32)
                zh.append(pltpu.async_copy(
                    zbuf.at[pl.ds(0, PLANE)],
                    o2.at[pl.ds((pair * 4 + pk) * PLANE, PLANE)], zsem))
            pk1 = (pe1 == 0).astype(jnp.int32)
            zh.append(pltpu.async_copy(
                zbuf.at[pl.ds(0, PLANE)],
                o1.at[pl.ds((pair * 2 + pk1) * PLANE, PLANE)], zsem))

        def body_jz(jz, carry):
            zs = szo + jz
            vz = (zs >= 0) & (zs < GSIZE)
            zcl = jnp.full((16,), jnp.clip(zs, 0, GSIZE - 1), jnp.int32)

            def body_j(j, c2):
                ys = syo + j
                vy = (ys >= 0) & (ys < GSIZE)
                ycl = jnp.full((16,), jnp.clip(ys, 0, GSIZE - 1), jnp.int32)
                scale = jnp.where(vz & vy, 1.0, 0.0).astype(jnp.float32)
                base = ((dz + jz) * SIZE + yc + j) * SIZE
                for i in range(4):
                    cx = sx + i * 16 + lane - 57
                    vxf = jnp.where((cx >= 0) & (cx < GSIZE), scale, 0.0)
                    cxc = jnp.clip(cx, 0, GSIZE - 1)
                    v = plsc.load_gather(
                        g3v, [(zcl * GSIZE + ycl) * 16 + cxc]) * vxf
                    pbig3[pl.ds(base + i * 16, 16)] = v
                return c2

            lax.fori_loop(0, GSIZE, body_j, 0)
            return carry

        lax.fori_loop(0, GSIZE, body_jz, 0)

        v2 = (zidx2 >= 0).astype(jnp.float32)
        v1 = (zidx1 >= 0).astype(jnp.float32)
        v0 = (zidx0 >= 0).astype(jnp.float32)

        def body_j2(j, carry):
            ys = syo + j
            vy = (ys >= 0) & (ys < GSIZE)
            ycl = jnp.full((16,), jnp.clip(ys, 0, GSIZE - 1), jnp.int32)
            yscale = jnp.where(vy, 1.0, 0.0).astype(jnp.float32)
            for r, vr in ((0, v0), (1, v1), (2, v2)):
                rcl = jnp.full((16,), r, jnp.int32)
                scale = yscale * vr
                base = (r * SIZE + yc + j) * SIZE
                for i in range(4):
                    cx = sx + i * 16 + lane - 57
                    vxf = jnp.where((cx >= 0) & (cx < GSIZE), scale, 0.0)
                    cxc = jnp.clip(cx, 0, GSIZE - 1)
                    v = plsc.load_gather(
                        g2v, [(rcl * GSIZE + ycl) * 16 + cxc]) * vxf
                    pbig2[pl.ds(base + i * 16, 16)] = v
            return carry

        lax.fori_loop(0, GSIZE, body_j2, 0)

        ph = [
            pltpu.async_copy(
                pbig3, o3.at[pl.ds((pair * 64 + bb) * PLANE, WIN * PLANE)],
                psem),
            pltpu.async_copy(
                pbig2.at[pl.ds(2 * PLANE, PLANE)],
                o2.at[pl.ds((pair * 4 + pe2) * PLANE, PLANE)], psem),
            pltpu.async_copy(
                pbig2.at[pl.ds(1 * PLANE, PLANE)],
                o1.at[pl.ds((pair * 2 + pe1) * PLANE, PLANE)], psem),
            pltpu.async_copy(
                pbig2.at[pl.ds(0, PLANE)],
                o0.at[pl.ds(pair * PLANE, PLANE)], psem),
        ]
        for h in ph:
            h.wait()

        def rz3(jz, carry):
            def rzj(j, c2):
                base = ((dz + jz) * SIZE + yc + j) * SIZE
                for i in range(4):
                    pbig3[pl.ds(base + i * 16, 16)] = zero16
                return c2
            lax.fori_loop(0, GSIZE, rzj, 0)
            return carry

        lax.fori_loop(0, GSIZE, rz3, 0)

        def rz2(j, carry):
            for r in range(3):
                base = (r * SIZE + yc + j) * SIZE
                for i in range(4):
                    pbig2[pl.ds(base + i * 16, 16)] = zero16
            return carry

        lax.fori_loop(0, GSIZE, rz2, 0)
        return zh

    zh_a = do_pair(wid)
    zh_b = do_pair(wid + 32)

    @pl.when(wid + 64 < NPAIR)
    def _():
        for h in do_pair(wid + 64, issue_zeros=False):
            h.wait()

    def extra_geom(pair):
        crow = coordsv[pl.ds(pair * 16, 16)]
        zz = crow[2]
        zc = jnp.clip(zz - 1 - PAD, 0, SIZE - GSIZE)
        bb4 = jnp.minimum(zc // 4, (SIZE - WIN) // 4)
        pe2 = jnp.maximum((zz * 4 + 63) // 64 - 1, 0)
        pe1 = jnp.maximum((zz * 2 + 63) // 64 - 1, 0)
        return bb4, pe2, pe1

    @pl.when((wid >= 8) & (wid < 16))
    def _():
        pair = 64 + wid - 8
        bb4, _pe2, _pe1 = extra_geom(pair)
        hs = []
        for j in range(6):
            blk = j + 5 * (j >= bb4).astype(jnp.int32)
            hs.append(pltpu.async_copy(
                zbuf,
                o3.at[pl.ds((pair * 64 + blk * 4) * PLANE, 4 * PLANE)],
                zsem))
        for h in hs:
            h.wait()

    @pl.when((wid >= 16) & (wid < 24))
    def _():
        pair = 64 + wid - 16
        bb4, pe2, pe1 = extra_geom(pair)
        hs = []
        for j in range(6, 11):
            blk = j + 5 * (j >= bb4).astype(jnp.int32)
            hs.append(pltpu.async_copy(
                zbuf,
                o3.at[pl.ds((pair * 64 + blk * 4) * PLANE, 4 * PLANE)],
                zsem))
        for k in range(3):
            pk = k + (k >= pe2).astype(jnp.int32)
            hs.append(pltpu.async_copy(
                zbuf.at[pl.ds(0, PLANE)],
                o2.at[pl.ds((pair * 4 + pk) * PLANE, PLANE)], zsem))
        pk1 = (pe1 == 0).astype(jnp.int32)
        hs.append(pltpu.async_copy(
            zbuf.at[pl.ds(0, PLANE)],
            o1.at[pl.ds((pair * 2 + pk1) * PLANE, PLANE)], zsem))
        for h in hs:
            h.wait()

    for h in zh_a + zh_b:
        h.wait()


@jax.jit
def kernel(coords, g0, g1, g2, g3):
    f32 = jnp.float32
    coords16 = jnp.zeros((NPAIR, 16), jnp.int32)
    coords16 = coords16.at[:, :3].set(coords.reshape(NPAIR, 3))
    g2s = jnp.zeros((3, GSIZE, 16), f32)
    g2s = g2s.at[:, :, :GSIZE].set(jnp.stack([g0[0], g1[0], g2[0]]).astype(f32))
    g3s = jnp.zeros((GSIZE, GSIZE, 16), f32)
    g3s = g3s.at[:, :, :GSIZE].set(g3.astype(f32))

    mesh = plsc.VectorSubcoreMesh(
        core_axis_name="c", subcore_axis_name="s",
        num_cores=NC, num_subcores=NS)
    out_type = [
        jax.ShapeDtypeStruct((NPAIR * 1 * PLANE,), f32),
        jax.ShapeDtypeStruct((NPAIR * 2 * PLANE,), f32),
        jax.ShapeDtypeStruct((NPAIR * 4 * PLANE,), f32),
        jax.ShapeDtypeStruct((NPAIR * 64 * PLANE,), f32),
    ]
    scratch = [
        pltpu.VMEM((NPAIR * 16,), jnp.int32),
        pltpu.VMEM((3 * GSIZE * 16,), f32),
        pltpu.VMEM((GSIZE * GSIZE * 16,), f32),
        pltpu.VMEM((4 * PLANE,), f32),
        pltpu.VMEM((3 * PLANE,), f32),
        pltpu.VMEM((WIN * PLANE,), f32),
        pltpu.SemaphoreType.DMA,
        pltpu.SemaphoreType.DMA,
    ]
    o0, o1, o2, o3 = pl.kernel(
        _sc_body, out_type=out_type, mesh=mesh, scratch_types=scratch,
        compiler_params=pltpu.CompilerParams(needs_layout_passes=False),
    )(coords16.reshape(-1), g2s.reshape(-1), g3s.reshape(-1),
      jnp.zeros((WIN * PLANE,), f32))
    return (
        o0.reshape(BATCH, PART, 1, SIZE, SIZE),
        o1.reshape(BATCH, PART, 2, SIZE, SIZE),
        o2.reshape(BATCH, PART, 4, SIZE, SIZE),
        o3.reshape(BATCH, PART, 64, SIZE, SIZE),
    )
